# Initial kernel scaffold; baseline (speedup 1.0000x reference)
import jax
import jax.numpy as jnp
from jax.experimental import pallas as pl


def kernel(fv, fe, fg, fv_pos, edge_index, params):
    raise NotImplementedError("write your pallas kernel here")


# speedup vs baseline: 6.6557x; 6.6557x over previous
import functools

import jax
import jax.numpy as jnp
from jax.experimental import pallas as pl
from jax.experimental.pallas import tpu as pltpu


def _gelu(x):
    return x * 0.5 * (1.0 + jax.lax.erf(x * 0.7071067811865476))


def _wt(p):
    return jnp.transpose(p['W'])


def _b(p, d):
    if 'b' in p:
        return p['b'].reshape(1, d)
    return jnp.zeros((1, d), jnp.float32)


def _k0_body(fg_ref, wg2v_ref, bg2v_ref, wga0_ref, bga0_ref, wga1_ref,
             wcg_ref, bcg_ref,
             gvec_ref, gateg_ref, cg_ref):
    fg = fg_ref[...]
    gvec = jnp.dot(fg, wg2v_ref[...], preferred_element_type=jnp.float32) + bg2v_ref[...]
    gvec_ref[...] = gvec[:, None, :]
    h = _gelu(jnp.dot(fg, wga0_ref[...], preferred_element_type=jnp.float32) + bga0_ref[...])
    gateg = jnp.dot(h, wga1_ref[...], preferred_element_type=jnp.float32)
    gateg_ref[...] = gateg[:, None, :]
    cg = jnp.dot(fg, wcg_ref[...], preferred_element_type=jnp.float32) + bcg_ref[...]
    cg_ref[...] = cg[:, None, :]


def _k1_body(fvp_ref, posp_ref, gvec_ref, wvu_ref, bvu_ref,
             fv1_ref, vu_ref, node8_ref, *, DV):
    fv = fvp_ref[0]
    gvec = gvec_ref[0]
    gsc = gvec[:, :DV]
    gsh = gvec[:, DV:]
    fv1 = fv * (1.0 + gsc) + gsh
    fv1_ref[0] = fv1
    vu_ref[0] = jnp.dot(fv1, wvu_ref[...], preferred_element_type=jnp.float32) + bvu_ref[...]
    pos = posp_ref[0]
    dist = jnp.sqrt(jnp.sum(pos * pos, axis=1, keepdims=True))
    lane = jax.lax.broadcasted_iota(jnp.int32, pos.shape, 1)
    node8_ref[0] = jnp.where(lane == 3, dist, pos)


def _k2_body(fe_ref, srcl_ref, dstl_ref, fv1_ref, vu_ref, node8_ref,
             gateg_ref,
             wg0_ref, bg0_ref, wg1_ref, bg1_ref, wg2_ref,
             wea0_ref, bea0_ref, wea1_ref, bea1_ref,
             wma_ref, wmb_ref, bm0_ref, wm1_ref, bm1_ref,
             fegeo_ref, agg_ref, rogeo_ref, *, NPP, TE, NH, DH, DV):
    t = pl.program_id(1)
    fe = fe_ref[0]
    srcl = srcl_ref[...]
    dstl = dstl_ref[...]
    niota = jax.lax.broadcasted_iota(jnp.int32, (TE, NPP), 1)
    oh_s = (srcl == niota).astype(jnp.float32)
    oh_d = (dstl == niota).astype(jnp.float32)

    node8 = node8_ref[0]
    n8s = jnp.dot(oh_s, node8, preferred_element_type=jnp.float32)
    n8d = jnp.dot(oh_d, node8, preferred_element_type=jnp.float32)
    lane8 = jax.lax.broadcasted_iota(jnp.int32, (TE, 8), 1)
    diff8 = jnp.where(lane8 < 3, n8s - n8d, 0.0)
    fed = jnp.sqrt(jnp.sum(diff8 * diff8, axis=1, keepdims=True))
    ds = n8s[:, 3:4]
    dd = n8d[:, 3:4]
    w0 = wg0_ref[...]
    h = fed * w0[0:1, :] + ds * w0[1:2, :] + dd * w0[2:3, :] + bg0_ref[...]
    h = _gelu(h)
    h = _gelu(jnp.dot(h, wg1_ref[...], preferred_element_type=jnp.float32) + bg1_ref[...])
    fe_geo = jnp.dot(h, wg2_ref[...], preferred_element_type=jnp.float32)
    fegeo_ref[0] = fe_geo

    fv1s = jnp.dot(oh_s, fv1_ref[0], preferred_element_type=jnp.float32)
    m0 = _gelu(jnp.dot(fv1s, wma_ref[...], preferred_element_type=jnp.float32)
               + jnp.dot(fe, wmb_ref[...], preferred_element_type=jnp.float32)
               + bm0_ref[...])
    msg = _gelu(jnp.dot(m0, wm1_ref[...], preferred_element_type=jnp.float32) + bm1_ref[...])

    vud = jnp.dot(oh_d, vu_ref[0], preferred_element_type=jnp.float32)
    fvs = vud[:, :DV]
    fvh = vud[:, DV:]
    m = _gelu(fvs * msg + fvh)

    g = _gelu(jnp.dot(fe_geo, wea0_ref[...], preferred_element_type=jnp.float32) + bea0_ref[...])
    gate = jnp.dot(g, wea1_ref[...], preferred_element_type=jnp.float32) + bea1_ref[...]
    gate = gate + gateg_ref[0]
    rep = (jax.lax.broadcasted_iota(jnp.int32, (NH, NH * DH), 1) // DH
           == jax.lax.broadcasted_iota(jnp.int32, (NH, NH * DH), 0)).astype(jnp.float32)
    gate_rep = jnp.dot(gate, rep, preferred_element_type=jnp.float32)

    contrib = m * gate_rep
    part = jax.lax.dot_general(oh_d, contrib, (((0,), (0,)), ((), ())),
                               preferred_element_type=jnp.float32)

    gsum = jnp.sum(fe_geo, axis=0, keepdims=True)
    gmin = jnp.min(fe_geo, axis=0, keepdims=True)
    gmax = jnp.max(fe_geo, axis=0, keepdims=True)
    zpad = jnp.zeros((5, fe_geo.shape[1]), jnp.float32)

    @pl.when(t == 0)
    def _():
        agg_ref[0] = part
        rogeo_ref[0] = jnp.concatenate([gsum, gmin, gmax, zpad], axis=0)

    @pl.when(t != 0)
    def _():
        agg_ref[0] += part
        old = rogeo_ref[0]
        s = old[0:1, :] + gsum
        mn = jnp.minimum(old[1:2, :], gmin)
        mx = jnp.maximum(old[2:3, :], gmax)
        rogeo_ref[0] = jnp.concatenate([s, mn, mx, zpad], axis=0)


def _k3_body(agg_ref, wproj_ref, bproj_ref, w1_ref, b1_ref, w2_ref, b2_ref,
             fvn_ref, a_ref, bv_ref, rov_ref, *, NP):
    agg = agg_ref[0]
    fvn = jnp.dot(agg, wproj_ref[...], preferred_element_type=jnp.float32) + bproj_ref[...]
    fvn_ref[0] = fvn
    a_ref[0] = jnp.dot(fvn, w1_ref[...], preferred_element_type=jnp.float32) + b1_ref[...]
    bv_ref[0] = jnp.dot(fvn, w2_ref[...], preferred_element_type=jnp.float32) + b2_ref[...]
    row = jax.lax.broadcasted_iota(jnp.int32, fvn.shape, 0)
    valid = row < NP
    big = jnp.float32(3.4e38)
    s = jnp.sum(jnp.where(valid, fvn, 0.0), axis=0, keepdims=True)
    mn = jnp.min(jnp.where(valid, fvn, big), axis=0, keepdims=True)
    mx = jnp.max(jnp.where(valid, fvn, -big), axis=0, keepdims=True)
    z = jnp.zeros((5, fvn.shape[1]), jnp.float32)
    rov_ref[0] = jnp.concatenate([s, mn, mx, z], axis=0)


def _k4_body(fe_ref, fegeo_ref, srcl_ref, dstl_ref, a_ref, bv_ref,
             node8_ref, posp_ref, cg_ref,
             wab_ref, wgeo_ref,
             w11_ref, b11_ref, w12_ref, b12_ref,
             w20_ref, b20_ref, w21_ref, b21_ref,
             wp0_ref, bp0_ref, wp1_ref, bp1_ref, wp2_ref,
             fen_ref, posagg_ref, roe_ref, *, NPP, TE, DE):
    t = pl.program_id(1)
    fe = fe_ref[0]
    fe_geo = fegeo_ref[0]
    srcl = srcl_ref[...]
    dstl = dstl_ref[...]
    niota = jax.lax.broadcasted_iota(jnp.int32, (TE, NPP), 1)
    oh_s = (srcl == niota).astype(jnp.float32)
    oh_d = (dstl == niota).astype(jnp.float32)

    ab = (jnp.dot(oh_s, a_ref[0], preferred_element_type=jnp.float32)
          * jnp.dot(oh_d, bv_ref[0], preferred_element_type=jnp.float32))
    h = _gelu(jnp.dot(ab, wab_ref[...], preferred_element_type=jnp.float32)
              + jnp.dot(fe_geo, wgeo_ref[...], preferred_element_type=jnp.float32)
              + cg_ref[0])
    h = _gelu(jnp.dot(h, w11_ref[...], preferred_element_type=jnp.float32) + b11_ref[...])
    eo = jnp.dot(h, w12_ref[...], preferred_element_type=jnp.float32) + b12_ref[...]
    esh = eo[:, :DE]
    esc = eo[:, DE:]

    f = _gelu(jnp.dot(fe, w20_ref[...], preferred_element_type=jnp.float32) + b20_ref[...])
    f = jnp.dot(f, w21_ref[...], preferred_element_type=jnp.float32) + b21_ref[...]
    fen = f * (esc + 1.0) + esh
    fen_ref[0] = fen

    q = _gelu(jnp.dot(fen, wp0_ref[...], preferred_element_type=jnp.float32) + bp0_ref[...])
    q = _gelu(jnp.dot(q, wp1_ref[...], preferred_element_type=jnp.float32) + bp1_ref[...])
    pm = jnp.dot(q, wp2_ref[...], preferred_element_type=jnp.float32)[:, 0:1]

    node8 = node8_ref[0]
    n8s = jnp.dot(oh_s, node8, preferred_element_type=jnp.float32)
    n8d = jnp.dot(oh_d, node8, preferred_element_type=jnp.float32)
    lane8 = jax.lax.broadcasted_iota(jnp.int32, (TE, 8), 1)
    diff8 = jnp.where(lane8 < 3, n8s - n8d, 0.0)
    fed = jnp.sqrt(jnp.sum(diff8 * diff8, axis=1, keepdims=True))
    pw8 = diff8 * (pm / (fed + 1.0))
    ppart = jax.lax.dot_general(oh_d, pw8, (((0,), (0,)), ((), ())),
                                preferred_element_type=jnp.float32)

    esum = jnp.sum(fen, axis=0, keepdims=True)
    emin = jnp.min(fen, axis=0, keepdims=True)
    emax = jnp.max(fen, axis=0, keepdims=True)
    z = jnp.zeros((5, fen.shape[1]), jnp.float32)

    @pl.when(t == 0)
    def _():
        posagg_ref[0] = posp_ref[0] + ppart
        roe_ref[0] = jnp.concatenate([esum, emin, emax, z], axis=0)

    @pl.when(t != 0)
    def _():
        posagg_ref[0] += ppart
        old = roe_ref[0]
        s = old[0:1, :] + esum
        mn = jnp.minimum(old[1:2, :], emin)
        mx = jnp.maximum(old[2:3, :], emax)
        roe_ref[0] = jnp.concatenate([s, mn, mx, z], axis=0)


def _k5_body(rov_ref, roe_ref, rogeo_ref, fg_ref,
             wvr1_ref, bvr1_ref, wvr2_ref, bvr2_ref, wvr3_ref, bvr3_ref,
             wer1_ref, ber1_ref, wer2_ref, ber2_ref, wer3_ref, ber3_ref,
             wgr1_ref, bgr1_ref, wgr2_ref, bgr2_ref, wgr3_ref, bgr3_ref,
             wg0a_ref, wg0b_ref, wg0c_ref, g0c_ref, wg1_ref, bg1_ref,
             wih_ref, bih_ref, whh_ref, bhh_ref,
             fgn_ref, *, NP, EP, DG):
    def dot(x, w):
        return jnp.dot(x, w[...], preferred_element_type=jnp.float32)

    rov = rov_ref[...]
    fv2g = (dot(rov[:, 0, :] / NP, wvr1_ref) + bvr1_ref[...]
            + dot(rov[:, 1, :], wvr2_ref) + bvr2_ref[...]
            + dot(rov[:, 2, :], wvr3_ref) + bvr3_ref[...])
    roe = roe_ref[...]
    fe2g = (dot(roe[:, 0, :] / EP, wer1_ref) + ber1_ref[...]
            + dot(roe[:, 1, :], wer2_ref) + ber2_ref[...]
            + dot(roe[:, 2, :], wer3_ref) + ber3_ref[...])
    rog = rogeo_ref[...]
    fg2g = (dot(rog[:, 0, :] / EP, wgr1_ref) + bgr1_ref[...]
            + dot(rog[:, 1, :], wgr2_ref) + bgr2_ref[...]
            + dot(rog[:, 2, :], wgr3_ref) + bgr3_ref[...])
    x = _gelu(dot(fv2g, wg0a_ref) + dot(fe2g, wg0b_ref) + dot(fg2g, wg0c_ref)
              + g0c_ref[...])
    x = dot(x, wg1_ref) + bg1_ref[...]
    fg = fg_ref[...]
    gi = dot(x, wih_ref) + bih_ref[...]
    gh = dot(fg, whh_ref) + bhh_ref[...]
    ir, iz, inn = gi[:, :DG], gi[:, DG:2 * DG], gi[:, 2 * DG:]
    hr, hz, hn = gh[:, :DG], gh[:, DG:2 * DG], gh[:, 2 * DG:]
    r = jax.nn.sigmoid(ir + hr)
    zg = jax.nn.sigmoid(iz + hz)
    n = jnp.tanh(inn + r * hn)
    fgn_ref[...] = (1.0 - zg) * n + zg * fg


def kernel(fv, fe, fg, fv_pos, edge_index, params):
    p = params
    N, DV = fv.shape
    E, DE = fe.shape
    B, DG = fg.shape
    NP = N // B
    EP = E // B
    NH = 4
    DH = 32
    DGEO = p['geo1']['W'].shape[0]
    NPP = ((NP + 127) // 128) * 128
    TE = min(EP, 2000)
    T = EP // TE
    f32 = jnp.float32

    src = edge_index[0].astype(jnp.int32)
    dst = edge_index[1].astype(jnp.int32)
    gid_e = (jnp.arange(E, dtype=jnp.int32) // EP) * NP
    srcl = (src - gid_e).reshape(E, 1)
    dstl = (dst - gid_e).reshape(E, 1)

    fvp = jnp.pad(fv.reshape(B, NP, DV), ((0, 0), (0, NPP - NP), (0, 0)))
    posp = jnp.pad(fv_pos.reshape(B, NP, 3), ((0, 0), (0, NPP - NP), (0, 5)))
    fe_r = fe.reshape(B, EP, DE)

    full = lambda s: pl.BlockSpec(s, lambda *_: tuple(0 for _ in s))

    w_eu10t = jnp.transpose(p['eu1_0']['W'])
    gvec, gateg, cg = pl.pallas_call(
        _k0_body,
        grid=(1,),
        in_specs=[full((B, DG)),
                  full((DG, 2 * DV)), full((1, 2 * DV)),
                  full((DG, DG)), full((1, DG)), full((DG, NH)),
                  full((DG, DE)), full((1, DE))],
        out_specs=[full((B, 1, 2 * DV)), full((B, 1, NH)), full((B, 1, DE))],
        out_shape=[jax.ShapeDtypeStruct((B, 1, 2 * DV), f32),
                   jax.ShapeDtypeStruct((B, 1, NH), f32),
                   jax.ShapeDtypeStruct((B, 1, DE), f32)],
    )(fg, _wt(p['lin_g2v']), _b(p['lin_g2v'], 2 * DV),
      _wt(p['ga0']), _b(p['ga0'], DG), _wt(p['ga1']),
      w_eu10t[DE + DGEO:], _b(p['eu1_0'], DE))

    gspec = lambda s: pl.BlockSpec(s, lambda g: (g,) + tuple(0 for _ in s[1:]))
    fv1, vu, node8 = pl.pallas_call(
        functools.partial(_k1_body, DV=DV),
        grid=(B,),
        in_specs=[gspec((1, NPP, DV)), gspec((1, NPP, 8)), gspec((1, 1, 2 * DV)),
                  full((DV, 2 * NH * DH)), full((1, 2 * NH * DH))],
        out_specs=[gspec((1, NPP, DV)), gspec((1, NPP, 2 * NH * DH)),
                   gspec((1, NPP, 8))],
        out_shape=[jax.ShapeDtypeStruct((B, NPP, DV), f32),
                   jax.ShapeDtypeStruct((B, NPP, 2 * NH * DH), f32),
                   jax.ShapeDtypeStruct((B, NPP, 8), f32)],
    )(fvp, posp, gvec, _wt(p['lin_v_upd']), _b(p['lin_v_upd'], 2 * NH * DH))

    espec = lambda s: pl.BlockSpec(s, lambda g, t: (g, t) + tuple(0 for _ in s[2:]))
    ispec = pl.BlockSpec((TE, 1), lambda g, t: (g * T + t, 0))
    gspec2 = lambda s: pl.BlockSpec(s, lambda g, t: (g,) + tuple(0 for _ in s[1:]))
    full2 = lambda s: pl.BlockSpec(s, lambda g, t: tuple(0 for _ in s))
    w_msg0t = jnp.transpose(p['vmsg0']['W'])
    fegeo, agg, rogeo = pl.pallas_call(
        functools.partial(_k2_body, NPP=NPP, TE=TE, NH=NH, DH=DH, DV=DV),
        grid=(B, T),
        in_specs=[espec((1, TE, DE)), ispec, ispec,
                  gspec2((1, NPP, DV)), gspec2((1, NPP, 2 * NH * DH)),
                  gspec2((1, NPP, 8)), gspec2((1, 1, NH)),
                  full2((8, DGEO)), full2((1, DGEO)),
                  full2((DGEO, DGEO)), full2((1, DGEO)), full2((DGEO, DGEO)),
                  full2((DGEO, DGEO)), full2((1, DGEO)),
                  full2((DGEO, NH)), full2((1, NH)),
                  full2((DV, DV)), full2((DE, DV)), full2((1, DV)),
                  full2((DV, NH * DH)), full2((1, NH * DH))],
        out_specs=[espec((1, TE, DGEO)), gspec2((1, NPP, DV)),
                   gspec2((1, 8, DGEO))],
        out_shape=[jax.ShapeDtypeStruct((B, EP, DGEO), f32),
                   jax.ShapeDtypeStruct((B, NPP, DV), f32),
                   jax.ShapeDtypeStruct((B, 8, DGEO), f32)],
    )(fe_r, srcl, dstl, fv1, vu, node8, gateg,
      jnp.pad(jnp.transpose(p['geo0']['W']), ((0, 5), (0, 0))), _b(p['geo0'], DGEO),
      _wt(p['geo1']), _b(p['geo1'], DGEO), _wt(p['geo2']),
      _wt(p['ega0']), _b(p['ega0'], DGEO),
      _wt(p['ega1']), _b(p['ega1'], NH),
      w_msg0t[:DV], w_msg0t[DV:], _b(p['vmsg0'], DV),
      _wt(p['vmsg1']), _b(p['vmsg1'], NH * DH))

    fvn, av, bv, rov = pl.pallas_call(
        functools.partial(_k3_body, NP=NP),
        grid=(B,),
        in_specs=[gspec((1, NPP, DV)),
                  full((NH * DH, DV)), full((1, DV)),
                  full((DV, DE)), full((1, DE)),
                  full((DV, DE)), full((1, DE))],
        out_specs=[gspec((1, NPP, DV)), gspec((1, NPP, DE)),
                   gspec((1, NPP, DE)), gspec((1, 8, DV))],
        out_shape=[jax.ShapeDtypeStruct((B, NPP, DV), f32),
                   jax.ShapeDtypeStruct((B, NPP, DE), f32),
                   jax.ShapeDtypeStruct((B, NPP, DE), f32),
                   jax.ShapeDtypeStruct((B, 8, DV), f32)],
    )(agg, _wt(p['v_upd_proj']), _b(p['v_upd_proj'], DV),
      _wt(p['v2e1']), _b(p['v2e1'], DE),
      _wt(p['v2e2']), _b(p['v2e2'], DE))

    wp2 = jnp.pad(jnp.transpose(p['pm2']['W']), ((0, 0), (0, DE - 1)))
    fen, posagg, roe = pl.pallas_call(
        functools.partial(_k4_body, NPP=NPP, TE=TE, DE=DE),
        grid=(B, T),
        in_specs=[espec((1, TE, DE)), espec((1, TE, DGEO)), ispec, ispec,
                  gspec2((1, NPP, DE)), gspec2((1, NPP, DE)),
                  gspec2((1, NPP, 8)), gspec2((1, NPP, 8)), gspec2((1, 1, DE)),
                  full2((DE, DE)), full2((DGEO, DE)),
                  full2((DE, DE)), full2((1, DE)),
                  full2((DE, 2 * DE)), full2((1, 2 * DE)),
                  full2((DE, DE)), full2((1, DE)),
                  full2((DE, DE)), full2((1, DE)),
                  full2((DE, DE)), full2((1, DE)),
                  full2((DE, DE)), full2((1, DE)),
                  full2((DE, DE))],
        out_specs=[espec((1, TE, DE)), gspec2((1, NPP, 8)), gspec2((1, 8, DE))],
        out_shape=[jax.ShapeDtypeStruct((B, EP, DE), f32),
                   jax.ShapeDtypeStruct((B, NPP, 8), f32),
                   jax.ShapeDtypeStruct((B, 8, DE), f32)],
    )(fe_r, fegeo, srcl, dstl, av, bv, node8, posp, cg,
      w_eu10t[:DE], w_eu10t[DE:DE + DGEO],
      _wt(p['eu1_1']), _b(p['eu1_1'], DE),
      _wt(p['eu1_2']), _b(p['eu1_2'], 2 * DE),
      _wt(p['eu2_0']), _b(p['eu2_0'], DE),
      _wt(p['eu2_1']), _b(p['eu2_1'], DE),
      _wt(p['pm0']), _b(p['pm0'], DE),
      _wt(p['pm1']), _b(p['pm1'], DE),
      wp2)

    wg0t = jnp.transpose(p['gmlp0']['W'])
    g0c = (NP * wg0t[2 * DG:2 * DG + 1, :] + _b(p['gmlp0'], DG))
    DH2 = DG // 2
    (fgn,) = pl.pallas_call(
        functools.partial(_k5_body, NP=float(NP), EP=float(EP), DG=DG),
        grid=(1,),
        in_specs=[full((B, 8, DV)), full((B, 8, DE)), full((B, 8, DGEO)),
                  full((B, DG)),
                  full((DV, DG)), full((1, DG)), full((DV, DG)), full((1, DG)),
                  full((DV, DG)), full((1, DG)),
                  full((DE, DH2)), full((1, DH2)), full((DE, DH2)), full((1, DH2)),
                  full((DE, DH2)), full((1, DH2)),
                  full((DGEO, DH2)), full((1, DH2)), full((DGEO, DH2)), full((1, DH2)),
                  full((DGEO, DH2)), full((1, DH2)),
                  full((DG, DG)), full((DH2, DG)), full((DH2, DG)), full((1, DG)),
                  full((DG, DG)), full((1, DG)),
                  full((DG, 3 * DG)), full((1, 3 * DG)),
                  full((DG, 3 * DG)), full((1, 3 * DG))],
        out_specs=[full((B, DG))],
        out_shape=[jax.ShapeDtypeStruct((B, DG), f32)],
    )(rov, roe, rogeo, fg,
      _wt(p['vr1']), _b(p['vr1'], DG), _wt(p['vr2']), _b(p['vr2'], DG),
      _wt(p['vr3']), _b(p['vr3'], DG),
      _wt(p['er1']), _b(p['er1'], DH2), _wt(p['er2']), _b(p['er2'], DH2),
      _wt(p['er3']), _b(p['er3'], DH2),
      _wt(p['gr1']), _b(p['gr1'], DH2), _wt(p['gr2']), _b(p['gr2'], DH2),
      _wt(p['gr3']), _b(p['gr3'], DH2),
      wg0t[:DG], wg0t[DG:DG + DH2], wg0t[DG + DH2:2 * DG], g0c,
      _wt(p['gmlp1']), _b(p['gmlp1'], DG),
      jnp.transpose(p['gru_Wih']), p['gru_bih'].reshape(1, 3 * DG),
      jnp.transpose(p['gru_Whh']), p['gru_bhh'].reshape(1, 3 * DG))

    fv_out = fvn[:, :NP, :].reshape(N, DV)
    fe_out = fen.reshape(E, DE)
    pos_out = posagg[:, :NP, :3].reshape(N, 3)
    return (fv_out, fe_out, fgn, pos_out)


# speedup vs baseline: 6.9377x; 1.0424x over previous
import functools

import jax
import jax.numpy as jnp
from jax.experimental import pallas as pl
from jax.experimental.pallas import tpu as pltpu


def _gelu(x):
    return x * 0.5 * (1.0 + jax.lax.erf(x * 0.7071067811865476))


def _dot(a, b):
    return jnp.dot(a.astype(jnp.bfloat16), b.astype(jnp.bfloat16),
                   preferred_element_type=jnp.float32)


def _dott(a, b):
    return jax.lax.dot_general(
        a.astype(jnp.bfloat16), b.astype(jnp.bfloat16),
        (((0,), (0,)), ((), ())), preferred_element_type=jnp.float32)


def _wt(p):
    return jnp.transpose(p['W'])


def _b(p, d):
    if 'b' in p:
        return p['b'].reshape(1, d)
    return jnp.zeros((1, d), jnp.float32)


def _k0_body(fg_ref, wg2v_ref, bg2v_ref, wga0_ref, bga0_ref, wga1_ref,
             wcg_ref, bcg_ref,
             gvec_ref, gateg_ref, cg_ref):
    fg = fg_ref[...]
    gvec = jnp.dot(fg, wg2v_ref[...], preferred_element_type=jnp.float32) + bg2v_ref[...]
    gvec_ref[...] = gvec[:, None, :]
    h = _gelu(jnp.dot(fg, wga0_ref[...], preferred_element_type=jnp.float32) + bga0_ref[...])
    gateg = jnp.dot(h, wga1_ref[...], preferred_element_type=jnp.float32)
    gateg_ref[...] = gateg[:, None, :]
    cg = jnp.dot(fg, wcg_ref[...], preferred_element_type=jnp.float32) + bcg_ref[...]
    cg_ref[...] = cg[:, None, :]


def _k1_body(fvp_ref, posp_ref, gvec_ref, wvu_ref, bvu_ref,
             fv1_ref, vu_ref, node8_ref, *, DV):
    fv = fvp_ref[0]
    gvec = gvec_ref[0]
    gsc = gvec[:, :DV]
    gsh = gvec[:, DV:]
    fv1 = fv * (1.0 + gsc) + gsh
    fv1_ref[0] = fv1.astype(jnp.bfloat16)
    vu = jnp.dot(fv1, wvu_ref[...], preferred_element_type=jnp.float32) + bvu_ref[...]
    vu_ref[0] = vu.astype(jnp.bfloat16)
    pos = posp_ref[0]
    dist = jnp.sqrt(jnp.sum(pos * pos, axis=1, keepdims=True))
    lane = jax.lax.broadcasted_iota(jnp.int32, pos.shape, 1)
    node8_ref[0] = jnp.where(lane == 3, dist, pos).astype(jnp.bfloat16)


def _k2_body(fe_ref, srcl_ref, dstl_ref, fv1_ref, vu_ref, node8_ref,
             gateg_ref,
             wg0_ref, bg0_ref, wg1_ref, bg1_ref, wg2_ref,
             wea0_ref, bea0_ref, wea1_ref, bea1_ref,
             wma_ref, wmb_ref, bm0_ref, wm1_ref, bm1_ref,
             fegeo_ref, agg_ref, rogeo_ref, *, NPP, TE, NH, DH, DV):
    t = pl.program_id(1)
    fe = fe_ref[0]
    srcl = srcl_ref[...]
    dstl = dstl_ref[...]
    niota = jax.lax.broadcasted_iota(jnp.int32, (TE, NPP), 1)
    oh_s = (srcl == niota).astype(jnp.bfloat16)
    oh_d = (dstl == niota).astype(jnp.bfloat16)

    node8 = node8_ref[0]
    n8s = _dot(oh_s, node8)
    n8d = _dot(oh_d, node8)
    lane8 = jax.lax.broadcasted_iota(jnp.int32, (TE, 8), 1)
    diff8 = jnp.where(lane8 < 3, n8s - n8d, 0.0)
    fed = jnp.sqrt(jnp.sum(diff8 * diff8, axis=1, keepdims=True))
    ds = n8s[:, 3:4]
    dd = n8d[:, 3:4]
    w0 = wg0_ref[...]
    h = fed * w0[0:1, :] + ds * w0[1:2, :] + dd * w0[2:3, :] + bg0_ref[...]
    h = _gelu(h)
    h = _gelu(_dot(h, wg1_ref[...]) + bg1_ref[...])
    fe_geo = _dot(h, wg2_ref[...])
    fegeo_ref[0] = fe_geo.astype(jnp.bfloat16)

    fv1s = _dot(oh_s, fv1_ref[0])
    m0 = _gelu(_dot(fv1s, wma_ref[...]) + _dot(fe, wmb_ref[...]) + bm0_ref[...])
    msg = _gelu(_dot(m0, wm1_ref[...]) + bm1_ref[...])

    vud = _dot(oh_d, vu_ref[0])
    fvs = vud[:, :DV]
    fvh = vud[:, DV:]
    m = _gelu(fvs * msg + fvh)

    g = _gelu(_dot(fe_geo, wea0_ref[...]) + bea0_ref[...])
    gate = _dot(g, wea1_ref[...]) + bea1_ref[...]
    gate = gate + gateg_ref[0]
    rep = (jax.lax.broadcasted_iota(jnp.int32, (NH, NH * DH), 1) // DH
           == jax.lax.broadcasted_iota(jnp.int32, (NH, NH * DH), 0)).astype(jnp.bfloat16)
    gate_rep = _dot(gate, rep)

    contrib = m * gate_rep
    part = _dott(oh_d, contrib)

    gsum = jnp.sum(fe_geo, axis=0, keepdims=True)
    gmin = jnp.min(fe_geo, axis=0, keepdims=True)
    gmax = jnp.max(fe_geo, axis=0, keepdims=True)
    zpad = jnp.zeros((5, fe_geo.shape[1]), jnp.float32)

    @pl.when(t == 0)
    def _():
        agg_ref[0] = part
        rogeo_ref[0] = jnp.concatenate([gsum, gmin, gmax, zpad], axis=0)

    @pl.when(t != 0)
    def _():
        agg_ref[0] += part
        old = rogeo_ref[0]
        s = old[0:1, :] + gsum
        mn = jnp.minimum(old[1:2, :], gmin)
        mx = jnp.maximum(old[2:3, :], gmax)
        rogeo_ref[0] = jnp.concatenate([s, mn, mx, zpad], axis=0)


def _k3_body(agg_ref, wproj_ref, bproj_ref, w1_ref, b1_ref, w2_ref, b2_ref,
             fvn_ref, a_ref, bv_ref, rov_ref, *, NP):
    agg = agg_ref[0]
    fvn = jnp.dot(agg, wproj_ref[...], preferred_element_type=jnp.float32) + bproj_ref[...]
    fvn_ref[0] = fvn
    a_ref[0] = (jnp.dot(fvn, w1_ref[...], preferred_element_type=jnp.float32)
                + b1_ref[...]).astype(jnp.bfloat16)
    bv_ref[0] = (jnp.dot(fvn, w2_ref[...], preferred_element_type=jnp.float32)
                 + b2_ref[...]).astype(jnp.bfloat16)
    row = jax.lax.broadcasted_iota(jnp.int32, fvn.shape, 0)
    valid = row < NP
    big = jnp.float32(3.4e38)
    s = jnp.sum(jnp.where(valid, fvn, 0.0), axis=0, keepdims=True)
    mn = jnp.min(jnp.where(valid, fvn, big), axis=0, keepdims=True)
    mx = jnp.max(jnp.where(valid, fvn, -big), axis=0, keepdims=True)
    z = jnp.zeros((5, fvn.shape[1]), jnp.float32)
    rov_ref[0] = jnp.concatenate([s, mn, mx, z], axis=0)


def _k4_body(fe_ref, fegeo_ref, srcl_ref, dstl_ref, a_ref, bv_ref,
             node8_ref, posp_ref, cg_ref,
             wab_ref, wgeo_ref,
             w11_ref, b11_ref, w12_ref, b12_ref,
             w20_ref, b20_ref, w21_ref, b21_ref,
             wp0_ref, bp0_ref, wp1_ref, bp1_ref, wp2_ref,
             fen_ref, posagg_ref, roe_ref, *, NPP, TE, DE):
    t = pl.program_id(1)
    fe = fe_ref[0]
    fe_geo = fegeo_ref[0]
    srcl = srcl_ref[...]
    dstl = dstl_ref[...]
    niota = jax.lax.broadcasted_iota(jnp.int32, (TE, NPP), 1)
    oh_s = (srcl == niota).astype(jnp.bfloat16)
    oh_d = (dstl == niota).astype(jnp.bfloat16)

    ab = _dot(oh_s, a_ref[0]) * _dot(oh_d, bv_ref[0])
    h = _gelu(_dot(ab, wab_ref[...]) + _dot(fe_geo, wgeo_ref[...]) + cg_ref[0])
    h = _gelu(_dot(h, w11_ref[...]) + b11_ref[...])
    eo = _dot(h, w12_ref[...]) + b12_ref[...]
    esh = eo[:, :DE]
    esc = eo[:, DE:]

    f = _gelu(_dot(fe, w20_ref[...]) + b20_ref[...])
    f = _dot(f, w21_ref[...]) + b21_ref[...]
    fen = f * (esc + 1.0) + esh
    fen_ref[0] = fen

    q = _gelu(_dot(fen, wp0_ref[...]) + bp0_ref[...])
    q = _gelu(_dot(q, wp1_ref[...]) + bp1_ref[...])
    pm = _dot(q, wp2_ref[...])[:, 0:1]

    node8 = node8_ref[0]
    n8s = _dot(oh_s, node8)
    n8d = _dot(oh_d, node8)
    lane8 = jax.lax.broadcasted_iota(jnp.int32, (TE, 8), 1)
    diff8 = jnp.where(lane8 < 3, n8s - n8d, 0.0)
    fed = jnp.sqrt(jnp.sum(diff8 * diff8, axis=1, keepdims=True))
    pw8 = diff8 * (pm / (fed + 1.0))
    ppart = _dott(oh_d, pw8)

    esum = jnp.sum(fen, axis=0, keepdims=True)
    emin = jnp.min(fen, axis=0, keepdims=True)
    emax = jnp.max(fen, axis=0, keepdims=True)
    z = jnp.zeros((5, fen.shape[1]), jnp.float32)

    @pl.when(t == 0)
    def _():
        posagg_ref[0] = posp_ref[0] + ppart
        roe_ref[0] = jnp.concatenate([esum, emin, emax, z], axis=0)

    @pl.when(t != 0)
    def _():
        posagg_ref[0] += ppart
        old = roe_ref[0]
        s = old[0:1, :] + esum
        mn = jnp.minimum(old[1:2, :], emin)
        mx = jnp.maximum(old[2:3, :], emax)
        roe_ref[0] = jnp.concatenate([s, mn, mx, z], axis=0)


def _k5_body(rov_ref, roe_ref, rogeo_ref, fg_ref,
             wvr1_ref, bvr1_ref, wvr2_ref, bvr2_ref, wvr3_ref, bvr3_ref,
             wer1_ref, ber1_ref, wer2_ref, ber2_ref, wer3_ref, ber3_ref,
             wgr1_ref, bgr1_ref, wgr2_ref, bgr2_ref, wgr3_ref, bgr3_ref,
             wg0a_ref, wg0b_ref, wg0c_ref, g0c_ref, wg1_ref, bg1_ref,
             wih_ref, bih_ref, whh_ref, bhh_ref,
             fgn_ref, *, NP, EP, DG):
    def dot(x, w):
        return jnp.dot(x, w[...], preferred_element_type=jnp.float32)

    rov = rov_ref[...]
    fv2g = (dot(rov[:, 0, :] / NP, wvr1_ref) + bvr1_ref[...]
            + dot(rov[:, 1, :], wvr2_ref) + bvr2_ref[...]
            + dot(rov[:, 2, :], wvr3_ref) + bvr3_ref[...])
    roe = roe_ref[...]
    fe2g = (dot(roe[:, 0, :] / EP, wer1_ref) + ber1_ref[...]
            + dot(roe[:, 1, :], wer2_ref) + ber2_ref[...]
            + dot(roe[:, 2, :], wer3_ref) + ber3_ref[...])
    rog = rogeo_ref[...]
    fg2g = (dot(rog[:, 0, :] / EP, wgr1_ref) + bgr1_ref[...]
            + dot(rog[:, 1, :], wgr2_ref) + bgr2_ref[...]
            + dot(rog[:, 2, :], wgr3_ref) + bgr3_ref[...])
    x = _gelu(dot(fv2g, wg0a_ref) + dot(fe2g, wg0b_ref) + dot(fg2g, wg0c_ref)
              + g0c_ref[...])
    x = dot(x, wg1_ref) + bg1_ref[...]
    fg = fg_ref[...]
    gi = dot(x, wih_ref) + bih_ref[...]
    gh = dot(fg, whh_ref) + bhh_ref[...]
    ir, iz, inn = gi[:, :DG], gi[:, DG:2 * DG], gi[:, 2 * DG:]
    hr, hz, hn = gh[:, :DG], gh[:, DG:2 * DG], gh[:, 2 * DG:]
    r = jax.nn.sigmoid(ir + hr)
    zg = jax.nn.sigmoid(iz + hz)
    n = jnp.tanh(inn + r * hn)
    fgn_ref[...] = (1.0 - zg) * n + zg * fg


def kernel(fv, fe, fg, fv_pos, edge_index, params):
    p = params
    N, DV = fv.shape
    E, DE = fe.shape
    B, DG = fg.shape
    NP = N // B
    EP = E // B
    NH = 4
    DH = 32
    DGEO = p['geo1']['W'].shape[0]
    NPP = ((NP + 127) // 128) * 128
    TE = min(EP, 4000)
    T = EP // TE
    f32 = jnp.float32

    src = edge_index[0].astype(jnp.int32)
    dst = edge_index[1].astype(jnp.int32)
    gid_e = (jnp.arange(E, dtype=jnp.int32) // EP) * NP
    srcl = (src - gid_e).reshape(E, 1)
    dstl = (dst - gid_e).reshape(E, 1)

    fvp = jnp.pad(fv.reshape(B, NP, DV), ((0, 0), (0, NPP - NP), (0, 0)))
    posp = jnp.pad(fv_pos.reshape(B, NP, 3), ((0, 0), (0, NPP - NP), (0, 5)))
    fe_r = fe.reshape(B, EP, DE)

    full = lambda s: pl.BlockSpec(s, lambda *_: tuple(0 for _ in s))

    w_eu10t = jnp.transpose(p['eu1_0']['W'])
    gvec, gateg, cg = pl.pallas_call(
        _k0_body,
        grid=(1,),
        in_specs=[full((B, DG)),
                  full((DG, 2 * DV)), full((1, 2 * DV)),
                  full((DG, DG)), full((1, DG)), full((DG, NH)),
                  full((DG, DE)), full((1, DE))],
        out_specs=[full((B, 1, 2 * DV)), full((B, 1, NH)), full((B, 1, DE))],
        out_shape=[jax.ShapeDtypeStruct((B, 1, 2 * DV), f32),
                   jax.ShapeDtypeStruct((B, 1, NH), f32),
                   jax.ShapeDtypeStruct((B, 1, DE), f32)],
    )(fg, _wt(p['lin_g2v']), _b(p['lin_g2v'], 2 * DV),
      _wt(p['ga0']), _b(p['ga0'], DG), _wt(p['ga1']),
      w_eu10t[DE + DGEO:], _b(p['eu1_0'], DE))

    gspec = lambda s: pl.BlockSpec(s, lambda g: (g,) + tuple(0 for _ in s[1:]))
    fv1, vu, node8 = pl.pallas_call(
        functools.partial(_k1_body, DV=DV),
        grid=(B,),
        in_specs=[gspec((1, NPP, DV)), gspec((1, NPP, 8)), gspec((1, 1, 2 * DV)),
                  full((DV, 2 * NH * DH)), full((1, 2 * NH * DH))],
        out_specs=[gspec((1, NPP, DV)), gspec((1, NPP, 2 * NH * DH)),
                   gspec((1, NPP, 8))],
        out_shape=[jax.ShapeDtypeStruct((B, NPP, DV), jnp.bfloat16),
                   jax.ShapeDtypeStruct((B, NPP, 2 * NH * DH), jnp.bfloat16),
                   jax.ShapeDtypeStruct((B, NPP, 8), jnp.bfloat16)],
    )(fvp, posp, gvec, _wt(p['lin_v_upd']), _b(p['lin_v_upd'], 2 * NH * DH))

    espec = lambda s: pl.BlockSpec(s, lambda g, t: (g, t) + tuple(0 for _ in s[2:]))
    ispec = pl.BlockSpec((TE, 1), lambda g, t: (g * T + t, 0))
    gspec2 = lambda s: pl.BlockSpec(s, lambda g, t: (g,) + tuple(0 for _ in s[1:]))
    full2 = lambda s: pl.BlockSpec(s, lambda g, t: tuple(0 for _ in s))
    w_msg0t = jnp.transpose(p['vmsg0']['W'])
    fegeo, agg, rogeo = pl.pallas_call(
        functools.partial(_k2_body, NPP=NPP, TE=TE, NH=NH, DH=DH, DV=DV),
        grid=(B, T),
        in_specs=[espec((1, TE, DE)), ispec, ispec,
                  gspec2((1, NPP, DV)), gspec2((1, NPP, 2 * NH * DH)),
                  gspec2((1, NPP, 8)), gspec2((1, 1, NH)),
                  full2((8, DGEO)), full2((1, DGEO)),
                  full2((DGEO, DGEO)), full2((1, DGEO)), full2((DGEO, DGEO)),
                  full2((DGEO, DGEO)), full2((1, DGEO)),
                  full2((DGEO, NH)), full2((1, NH)),
                  full2((DV, DV)), full2((DE, DV)), full2((1, DV)),
                  full2((DV, NH * DH)), full2((1, NH * DH))],
        out_specs=[espec((1, TE, DGEO)), gspec2((1, NPP, DV)),
                   gspec2((1, 8, DGEO))],
        out_shape=[jax.ShapeDtypeStruct((B, EP, DGEO), jnp.bfloat16),
                   jax.ShapeDtypeStruct((B, NPP, DV), f32),
                   jax.ShapeDtypeStruct((B, 8, DGEO), f32)],
    )(fe_r, srcl, dstl, fv1, vu, node8, gateg,
      jnp.pad(jnp.transpose(p['geo0']['W']), ((0, 5), (0, 0))), _b(p['geo0'], DGEO),
      _wt(p['geo1']), _b(p['geo1'], DGEO), _wt(p['geo2']),
      _wt(p['ega0']), _b(p['ega0'], DGEO),
      _wt(p['ega1']), _b(p['ega1'], NH),
      w_msg0t[:DV], w_msg0t[DV:], _b(p['vmsg0'], DV),
      _wt(p['vmsg1']), _b(p['vmsg1'], NH * DH))

    fvn, av, bv, rov = pl.pallas_call(
        functools.partial(_k3_body, NP=NP),
        grid=(B,),
        in_specs=[gspec((1, NPP, DV)),
                  full((NH * DH, DV)), full((1, DV)),
                  full((DV, DE)), full((1, DE)),
                  full((DV, DE)), full((1, DE))],
        out_specs=[gspec((1, NPP, DV)), gspec((1, NPP, DE)),
                   gspec((1, NPP, DE)), gspec((1, 8, DV))],
        out_shape=[jax.ShapeDtypeStruct((B, NPP, DV), f32),
                   jax.ShapeDtypeStruct((B, NPP, DE), jnp.bfloat16),
                   jax.ShapeDtypeStruct((B, NPP, DE), jnp.bfloat16),
                   jax.ShapeDtypeStruct((B, 8, DV), f32)],
    )(agg, _wt(p['v_upd_proj']), _b(p['v_upd_proj'], DV),
      _wt(p['v2e1']), _b(p['v2e1'], DE),
      _wt(p['v2e2']), _b(p['v2e2'], DE))

    wp2 = jnp.pad(jnp.transpose(p['pm2']['W']), ((0, 0), (0, DE - 1)))
    fen, posagg, roe = pl.pallas_call(
        functools.partial(_k4_body, NPP=NPP, TE=TE, DE=DE),
        grid=(B, T),
        in_specs=[espec((1, TE, DE)), espec((1, TE, DGEO)), ispec, ispec,
                  gspec2((1, NPP, DE)), gspec2((1, NPP, DE)),
                  gspec2((1, NPP, 8)), gspec2((1, NPP, 8)), gspec2((1, 1, DE)),
                  full2((DE, DE)), full2((DGEO, DE)),
                  full2((DE, DE)), full2((1, DE)),
                  full2((DE, 2 * DE)), full2((1, 2 * DE)),
                  full2((DE, DE)), full2((1, DE)),
                  full2((DE, DE)), full2((1, DE)),
                  full2((DE, DE)), full2((1, DE)),
                  full2((DE, DE)), full2((1, DE)),
                  full2((DE, DE))],
        out_specs=[espec((1, TE, DE)), gspec2((1, NPP, 8)), gspec2((1, 8, DE))],
        out_shape=[jax.ShapeDtypeStruct((B, EP, DE), f32),
                   jax.ShapeDtypeStruct((B, NPP, 8), f32),
                   jax.ShapeDtypeStruct((B, 8, DE), f32)],
    )(fe_r, fegeo, srcl, dstl, av, bv, node8, posp, cg,
      w_eu10t[:DE], w_eu10t[DE:DE + DGEO],
      _wt(p['eu1_1']), _b(p['eu1_1'], DE),
      _wt(p['eu1_2']), _b(p['eu1_2'], 2 * DE),
      _wt(p['eu2_0']), _b(p['eu2_0'], DE),
      _wt(p['eu2_1']), _b(p['eu2_1'], DE),
      _wt(p['pm0']), _b(p['pm0'], DE),
      _wt(p['pm1']), _b(p['pm1'], DE),
      wp2)

    wg0t = jnp.transpose(p['gmlp0']['W'])
    g0c = (NP * wg0t[2 * DG:2 * DG + 1, :] + _b(p['gmlp0'], DG))
    DH2 = DG // 2
    (fgn,) = pl.pallas_call(
        functools.partial(_k5_body, NP=float(NP), EP=float(EP), DG=DG),
        grid=(1,),
        in_specs=[full((B, 8, DV)), full((B, 8, DE)), full((B, 8, DGEO)),
                  full((B, DG)),
                  full((DV, DG)), full((1, DG)), full((DV, DG)), full((1, DG)),
                  full((DV, DG)), full((1, DG)),
                  full((DE, DH2)), full((1, DH2)), full((DE, DH2)), full((1, DH2)),
                  full((DE, DH2)), full((1, DH2)),
                  full((DGEO, DH2)), full((1, DH2)), full((DGEO, DH2)), full((1, DH2)),
                  full((DGEO, DH2)), full((1, DH2)),
                  full((DG, DG)), full((DH2, DG)), full((DH2, DG)), full((1, DG)),
                  full((DG, DG)), full((1, DG)),
                  full((DG, 3 * DG)), full((1, 3 * DG)),
                  full((DG, 3 * DG)), full((1, 3 * DG))],
        out_specs=[full((B, DG))],
        out_shape=[jax.ShapeDtypeStruct((B, DG), f32)],
    )(rov, roe, rogeo, fg,
      _wt(p['vr1']), _b(p['vr1'], DG), _wt(p['vr2']), _b(p['vr2'], DG),
      _wt(p['vr3']), _b(p['vr3'], DG),
      _wt(p['er1']), _b(p['er1'], DH2), _wt(p['er2']), _b(p['er2'], DH2),
      _wt(p['er3']), _b(p['er3'], DH2),
      _wt(p['gr1']), _b(p['gr1'], DH2), _wt(p['gr2']), _b(p['gr2'], DH2),
      _wt(p['gr3']), _b(p['gr3'], DH2),
      wg0t[:DG], wg0t[DG:DG + DH2], wg0t[DG + DH2:2 * DG], g0c,
      _wt(p['gmlp1']), _b(p['gmlp1'], DG),
      jnp.transpose(p['gru_Wih']), p['gru_bih'].reshape(1, 3 * DG),
      jnp.transpose(p['gru_Whh']), p['gru_bhh'].reshape(1, 3 * DG))

    fv_out = fvn[:, :NP, :].reshape(N, DV)
    fe_out = fen.reshape(E, DE)
    pos_out = posagg[:, :NP, :3].reshape(N, 3)
    return (fv_out, fe_out, fgn, pos_out)
